# baseline (device time: 12268 ns/iter reference)
import jax
import jax.numpy as jnp
from jax import lax
from jax.experimental import pallas as pl
from jax.experimental.pallas import tpu as pltpu

C = 8


def kernel(x):
    m, n = x.shape
    half = m // 2
    ch = half // C

    def body(x_ref, out_ref, send_x, recv_x, sx_sems, rx_sems):
        my_x = lax.axis_index("x")
        my_y = lax.axis_index("y")
        my_z = lax.axis_index("z")
        xpeer = (1 - my_x, my_y, my_z)

        barrier = pltpu.get_barrier_semaphore()
        pl.semaphore_signal(
            barrier, inc=1, device_id=xpeer, device_id_type=pl.DeviceIdType.MESH
        )
        pl.semaphore_wait(barrier, 1)

        base = my_z * half

        rdmas_a = []
        for c in range(C):
            r0 = c * ch
            send_x[pl.ds(r0, ch), :] = x_ref[pl.ds(base + r0, ch), :].astype(
                jnp.bfloat16
            )
            rd = pltpu.make_async_remote_copy(
                src_ref=send_x.at[pl.ds(r0, ch), :],
                dst_ref=recv_x.at[pl.ds(r0, ch), :],
                send_sem=sx_sems.at[c],
                recv_sem=rx_sems.at[c],
                device_id=xpeer,
                device_id_type=pl.DeviceIdType.MESH,
            )
            rd.start()
            rdmas_a.append(rd)

        for c in range(C):
            r0 = c * ch
            rdmas_a[c].wait_recv()
            out_ref[pl.ds(base + r0, ch), :] = (
                send_x[pl.ds(r0, ch), :] + recv_x[pl.ds(r0, ch), :]
            )
        for c in range(C):
            rdmas_a[c].wait_send()

    return pl.pallas_call(
        body,
        out_shape=jax.ShapeDtypeStruct((m, n), jnp.bfloat16),
        in_specs=[pl.BlockSpec(memory_space=pltpu.VMEM)],
        out_specs=pl.BlockSpec(memory_space=pltpu.VMEM),
        scratch_shapes=[
            pltpu.VMEM((half, n), jnp.bfloat16),
            pltpu.VMEM((half, n), jnp.bfloat16),
            pltpu.SemaphoreType.DMA((C,)),
            pltpu.SemaphoreType.DMA((C,)),
        ],
        compiler_params=pltpu.CompilerParams(collective_id=0),
    )(x)
